# Initial kernel scaffold; baseline (speedup 1.0000x reference)
import jax
import jax.numpy as jnp
from jax.experimental import pallas as pl


def kernel(batch_graph, adj, W1, a_src1, a_dst1, b1, W2, a_src2, a_dst2, b2, W3, a_src3, a_dst3, b3):
    raise NotImplementedError("write your pallas kernel here")


# speedup vs baseline: 2101.1147x; 2101.1147x over previous
import jax
import jax.numpy as jnp
from jax.experimental import pallas as pl


def _gat_layer(x, mask, neg_mask_bias, w_t, a_s, a_d, bias):
    h = jnp.dot(x, w_t, preferred_element_type=jnp.float32)
    alpha_s = jnp.sum(h * a_s, axis=1, keepdims=True)
    alpha_d = jnp.sum(h * a_d, axis=1, keepdims=True)
    e = alpha_s + alpha_d.T
    e = jnp.where(e >= 0.0, e, 0.2 * e)
    e_masked = jnp.where(mask, e, neg_mask_bias)
    emax = jnp.max(e_masked, axis=0, keepdims=True)
    emax = jnp.where(emax > neg_mask_bias, emax, 0.0)
    p = jnp.where(mask, jnp.exp(e - emax), 0.0)
    denom = jnp.sum(p, axis=0, keepdims=True)
    a = p / (denom + 1e-16)
    out = jnp.dot(a.T, h, preferred_element_type=jnp.float32)
    return out + bias


def _gat3_kernel(x_ref, adj_ref, w1_ref, as1_ref, ad1_ref, b1_ref,
                 w2_ref, as2_ref, ad2_ref, b2_ref,
                 w3_ref, as3_ref, ad3_ref, b3_ref, out_ref):
    x = x_ref[0]
    mask = adj_ref[0] > 0.0
    neg = jnp.float32(-1e30)
    x = _gat_layer(x, mask, neg, w1_ref[...], as1_ref[...], ad1_ref[...], b1_ref[...])
    x = _gat_layer(x, mask, neg, w2_ref[...], as2_ref[...], ad2_ref[...], b2_ref[...])
    x = _gat_layer(x, mask, neg, w3_ref[...], as3_ref[...], ad3_ref[...], b3_ref[...])
    out_ref[0] = x


def kernel(batch_graph, adj, W1, a_src1, a_dst1, b1, W2, a_src2, a_dst2, b2,
           W3, a_src3, a_dst3, b3):
    B, N, in_dim = batch_graph.shape
    hid = W1.shape[0]
    out_dim = W3.shape[0]

    def vec2d(v):
        return v.reshape(1, -1)

    full = lambda shape: pl.BlockSpec(shape, lambda b: (0,) * len(shape))
    batched = lambda shape: pl.BlockSpec((1,) + shape, lambda b: (b, 0, 0))

    return pl.pallas_call(
        _gat3_kernel,
        grid=(B,),
        in_specs=[
            batched((N, in_dim)),
            batched((N, N)),
            full((in_dim, hid)),
            full((1, hid)), full((1, hid)), full((1, hid)),
            full((hid, hid)),
            full((1, hid)), full((1, hid)), full((1, hid)),
            full((hid, out_dim)),
            full((1, out_dim)), full((1, out_dim)), full((1, out_dim)),
        ],
        out_specs=batched((N, out_dim)),
        out_shape=jax.ShapeDtypeStruct((B, N, out_dim), jnp.float32),
    )(batch_graph, adj,
      W1.T, vec2d(a_src1), vec2d(a_dst1), vec2d(b1),
      W2.T, vec2d(a_src2), vec2d(a_dst2), vec2d(b2),
      W3.T, vec2d(a_src3), vec2d(a_dst3), vec2d(b3))
